# baseline (speedup 1.0000x reference)
import jax
import jax.numpy as jnp
from jax.experimental import pallas as pl
from jax.experimental.pallas import tpu as pltpu

_SIGN = -2147483648
_IMAX = 2147483647


def _key_kernel(x_ref, e_ref, key_ref):
    t = jnp.exp(x_ref[...])
    bits = jax.lax.bitcast_convert_type(t, jnp.int32)
    key = (bits << 1) ^ _SIGN
    key_ref[...] = jnp.where(e_ref[...] > 0.0, key, _IMAX)


def _count_kernel(xb_ref, eb_ref, sb_ref, ka_ref, sa_ref, out_ref):
    rows, cols = xb_ref.shape
    n = rows * cols
    t_b = jnp.exp(xb_ref[...])
    bits_b = jax.lax.bitcast_convert_type(t_b, jnp.int32)
    e_b = eb_ref[...].astype(jnp.int32)
    key_b = ((bits_b << 1) | (1 - e_b)) ^ _SIGN
    s_b = sb_ref[...]

    def body(a, carry):
        acc1, acc2 = carry
        ka = ka_ref[a]
        sa = sa_ref[a]

        def work(c):
            a1, a2 = c
            cmp = ka < key_b
            conm = s_b < sa
            tiem = jnp.abs(s_b - sa) <= 1e-8
            a1 = a1 + jnp.where(cmp, jnp.where(conm, 65537, 1), 0)
            a2 = a2 + jnp.where(cmp & tiem, 1, 0)
            return a1, a2

        return jax.lax.cond(ka != _IMAX, work, lambda c: c, (acc1, acc2))

    zeros = jnp.zeros((rows, cols), jnp.int32)
    acc1, acc2 = jax.lax.fori_loop(0, n, body, (zeros, zeros), unroll=4)

    tot = jnp.sum(acc1 & 65535)
    con = jnp.sum(acc1 >> 16)
    tie = jnp.sum(acc2)

    tie_f = tie.astype(jnp.float32)
    tot_f = tot.astype(jnp.float32)
    disc_f = (tot - con - tie).astype(jnp.float32)
    loss = 1.0 - (disc_f + 0.5 * tie_f) / (tot_f + 1e-7)
    out_ref[...] = jnp.broadcast_to(loss, (1, 1))


def kernel(event_indicator, event_time, estimate):
    x = jnp.asarray(event_time, jnp.float32).reshape(-1)
    s = jnp.asarray(estimate, jnp.float32).reshape(-1)
    e = jnp.asarray(event_indicator).astype(jnp.float32).reshape(-1)
    n = x.shape[0]
    rows, cols = 8, n // 8

    key_a = pl.pallas_call(
        _key_kernel,
        out_shape=jax.ShapeDtypeStruct((1, n), jnp.int32),
    )(x.reshape(1, n), e.reshape(1, n))

    out = pl.pallas_call(
        _count_kernel,
        in_specs=[
            pl.BlockSpec(memory_space=pltpu.VMEM),
            pl.BlockSpec(memory_space=pltpu.VMEM),
            pl.BlockSpec(memory_space=pltpu.VMEM),
            pl.BlockSpec(memory_space=pltpu.SMEM),
            pl.BlockSpec(memory_space=pltpu.SMEM),
        ],
        out_shape=jax.ShapeDtypeStruct((1, 1), jnp.float32),
    )(
        x.reshape(rows, cols), e.reshape(rows, cols), s.reshape(rows, cols),
        key_a.reshape(n), s.reshape(n),
    )
    return out[0, 0]


# speedup vs baseline: 1.1982x; 1.1982x over previous
import jax
import jax.numpy as jnp
from jax.experimental import pallas as pl
from jax.experimental.pallas import tpu as pltpu

_SIGN = -2147483648
_IMAX = 2147483647


def _key_kernel(x_ref, e_ref, key_ref):
    t = jnp.exp(x_ref[...])
    bits = jax.lax.bitcast_convert_type(t, jnp.int32)
    key = (bits << 1) ^ _SIGN
    key_ref[...] = jnp.where(e_ref[...] > 0.0, key, _IMAX)


def _count_kernel(xb_ref, eb_ref, sb_ref, ka_ref, sa_ref, out_ref):
    rows, cols = xb_ref.shape
    n = rows * cols
    t_b = jnp.exp(xb_ref[...])
    bits_b = jax.lax.bitcast_convert_type(t_b, jnp.int32)
    e_b = eb_ref[...].astype(jnp.int32)
    key_b = ((bits_b << 1) | (1 - e_b)) ^ _SIGN
    s_b = sb_ref[...]

    half = n // 2
    base = pl.program_id(0) * half

    def body(i, carry):
        acc1, acc2 = carry
        a = base + i
        ka = ka_ref[a]
        sa = sa_ref[a]
        cmp = ka < key_b
        conm = s_b < sa
        tiem = jnp.abs(s_b - sa) <= 1e-8
        acc1 = acc1 + jnp.where(cmp, jnp.where(conm, 65537, 1), 0)
        acc2 = acc2 + jnp.where(cmp & tiem, 1, 0)
        return acc1, acc2

    zeros = jnp.zeros((rows, cols), jnp.int32)
    acc1, acc2 = jax.lax.fori_loop(0, half, body, (zeros, zeros), unroll=8)

    out_ref[0, 0, 0] = jnp.sum(acc1 & 65535)
    out_ref[0, 0, 1] = jnp.sum(acc1 >> 16)
    out_ref[0, 0, 2] = jnp.sum(acc2)


def kernel(event_indicator, event_time, estimate):
    x = jnp.asarray(event_time, jnp.float32).reshape(-1)
    s = jnp.asarray(estimate, jnp.float32).reshape(-1)
    e = jnp.asarray(event_indicator).astype(jnp.float32).reshape(-1)
    n = x.shape[0]
    rows, cols = 8, n // 8

    key_a = pl.pallas_call(
        _key_kernel,
        out_shape=jax.ShapeDtypeStruct((1, n), jnp.int32),
    )(x.reshape(1, n), e.reshape(1, n))

    counts = pl.pallas_call(
        _count_kernel,
        grid=(2,),
        in_specs=[
            pl.BlockSpec((rows, cols), lambda p: (0, 0), memory_space=pltpu.VMEM),
            pl.BlockSpec((rows, cols), lambda p: (0, 0), memory_space=pltpu.VMEM),
            pl.BlockSpec((rows, cols), lambda p: (0, 0), memory_space=pltpu.VMEM),
            pl.BlockSpec(memory_space=pltpu.SMEM),
            pl.BlockSpec(memory_space=pltpu.SMEM),
        ],
        out_specs=pl.BlockSpec((1, 1, 3), lambda p: (p, 0, 0), memory_space=pltpu.SMEM),
        out_shape=jax.ShapeDtypeStruct((2, 1, 3), jnp.int32),
        compiler_params=pltpu.CompilerParams(
            dimension_semantics=("parallel",),
        ),
    )(
        x.reshape(rows, cols), e.reshape(rows, cols), s.reshape(rows, cols),
        key_a.reshape(n), s.reshape(n),
    )
    tot, con, tie = (counts.sum(axis=(0, 1)).astype(jnp.float32)[i] for i in range(3))
    disc = tot - con - tie
    loss = 1.0 - (disc + 0.5 * tie) / (tot + 1e-7)
    return loss


# speedup vs baseline: 1.2803x; 1.0685x over previous
import jax
import jax.numpy as jnp
from jax.experimental import pallas as pl
from jax.experimental.pallas import tpu as pltpu

_SIGN = -2147483648
_IMAX = 2147483647


def _key_kernel(x_ref, e_ref, key_ref):
    t = jnp.exp(x_ref[...])
    bits = jax.lax.bitcast_convert_type(t, jnp.int32)
    key = (bits << 1) ^ _SIGN
    key_ref[...] = jnp.where(e_ref[...] > 0.0, key, _IMAX)


def _count_kernel(xb_ref, eb_ref, sb_ref, ka_ref, sa_ref, out_ref,
                  kc_ref, sc_ref):
    rows, cols = xb_ref.shape
    n = rows * cols
    t_b = jnp.exp(xb_ref[...])
    bits_b = jax.lax.bitcast_convert_type(t_b, jnp.int32)
    e_b = eb_ref[...].astype(jnp.int32)
    key_b = ((bits_b << 1) | (1 - e_b)) ^ _SIGN
    s_b = sb_ref[...]

    def compact(a, cnt):
        ka = ka_ref[a]
        kc_ref[cnt] = ka
        sc_ref[cnt] = sa_ref[a]
        return cnt + jnp.where(ka != _IMAX, 1, 0).astype(jnp.int32)

    m = jax.lax.fori_loop(0, n, compact, jnp.int32(0), unroll=4)

    for j in range(8):
        kc_ref[m + j] = _IMAX

    def body(i, carry):
        acc1, acc2 = carry
        base = i * 8
        for j in range(8):
            ka = kc_ref[base + j]
            sa = sc_ref[base + j]
            cmp = ka < key_b
            conm = s_b < sa
            tiem = jnp.abs(s_b - sa) <= 1e-8
            acc1 = acc1 + jnp.where(cmp, jnp.where(conm, 65537, 1), 0)
            acc2 = acc2 + jnp.where(cmp & tiem, 1, 0)
        return acc1, acc2

    zeros = jnp.zeros((rows, cols), jnp.int32)
    acc1, acc2 = jax.lax.fori_loop(0, (m + 7) // 8, body, (zeros, zeros))

    tot = jnp.sum(acc1 & 65535)
    con = jnp.sum(acc1 >> 16)
    tie = jnp.sum(acc2)

    tie_f = tie.astype(jnp.float32)
    tot_f = tot.astype(jnp.float32)
    disc_f = (tot - con - tie).astype(jnp.float32)
    loss = 1.0 - (disc_f + 0.5 * tie_f) / (tot_f + 1e-7)
    out_ref[...] = jnp.broadcast_to(loss, (1, 1))


def kernel(event_indicator, event_time, estimate):
    x = jnp.asarray(event_time, jnp.float32).reshape(-1)
    s = jnp.asarray(estimate, jnp.float32).reshape(-1)
    e = jnp.asarray(event_indicator).astype(jnp.float32).reshape(-1)
    n = x.shape[0]
    rows, cols = 8, n // 8

    key_a = pl.pallas_call(
        _key_kernel,
        out_shape=jax.ShapeDtypeStruct((1, n), jnp.int32),
    )(x.reshape(1, n), e.reshape(1, n))

    out = pl.pallas_call(
        _count_kernel,
        in_specs=[
            pl.BlockSpec(memory_space=pltpu.VMEM),
            pl.BlockSpec(memory_space=pltpu.VMEM),
            pl.BlockSpec(memory_space=pltpu.VMEM),
            pl.BlockSpec(memory_space=pltpu.SMEM),
            pl.BlockSpec(memory_space=pltpu.SMEM),
        ],
        out_shape=jax.ShapeDtypeStruct((1, 1), jnp.float32),
        scratch_shapes=[
            pltpu.SMEM((n + 8,), jnp.int32),
            pltpu.SMEM((n + 8,), jnp.float32),
        ],
    )(
        x.reshape(rows, cols), e.reshape(rows, cols), s.reshape(rows, cols),
        key_a.reshape(n), s.reshape(n),
    )
    return out[0, 0]
